# baseline (speedup 1.0000x reference)
import functools
import jax
import jax.numpy as jnp
from jax import lax
from jax.experimental import pallas as pl
from jax.experimental.pallas import tpu as pltpu
from jax.experimental.pallas import tpu_sc as plsc

V = 100000
W = 4
E = 2
L = 8
R = 16
NROWBLK = 8
CW = 512
NCHUNK = V // CW
TAILW = V - NCHUNK * CW
CWE = 2048
NCHUNKE = V // CWE
U1 = 5
U2 = 4


def _merge_top2(carry, c, idx):
    m1, i1, m2, i2 = carry
    cm1 = jnp.max(c, axis=-1, keepdims=True)
    ci1 = jnp.min(jnp.where(c == cm1, idx, V), axis=-1, keepdims=True)
    cmask = jnp.where(idx == ci1, -jnp.inf, c)
    cm2 = jnp.max(cmask, axis=-1, keepdims=True)
    ci2 = jnp.min(jnp.where(cmask == cm2, idx, V), axis=-1, keepdims=True)
    chunk_wins = cm1 > m1
    nm1 = jnp.where(chunk_wins, cm1, m1)
    ni1 = jnp.where(chunk_wins, ci1, i1)
    la_v = jnp.where(chunk_wins, m1, cm1)
    la_i = jnp.where(chunk_wins, i1, ci1)
    lb_v = jnp.where(chunk_wins, cm2, m2)
    lb_i = jnp.where(chunk_wins, ci2, i2)
    b_better = (lb_v > la_v) | ((lb_v == la_v) & (lb_i < la_i))
    nm2 = jnp.where(b_better, lb_v, la_v)
    ni2 = jnp.where(b_better, lb_i, la_i)
    return nm1, ni1, nm2, ni2


def _stats_kernel(x_ref, m1_ref, i1_ref, m2_ref, i2_ref, lse_ref):
    base_iota = lax.broadcasted_iota(jnp.int32, (R, CW), 1)

    def p1(j, carry):
        M1, I1, M2, I2 = carry
        cs = [x_ref[:, pl.ds(pl.multiple_of(j * (U1 * CW) + u * CW, CW), CW)]
              for u in range(U1)]
        for u in range(U1):
            c = cs[u]
            idx = base_iota + (j * U1 + u) * CW
            gt = c > M1
            gt2 = c > M2
            M2 = jnp.where(gt, M1, jnp.where(gt2, c, M2))
            I2 = jnp.where(gt, I1, jnp.where(gt2, idx, I2))
            M1 = jnp.where(gt, c, M1)
            I1 = jnp.where(gt, idx, I1)
        return M1, I1, M2, I2

    init = (jnp.full((R, CW), -jnp.inf, jnp.float32),
            jnp.full((R, CW), V, jnp.int32),
            jnp.full((R, CW), -jnp.inf, jnp.float32),
            jnp.full((R, CW), V, jnp.int32))
    M1, I1, M2, I2 = lax.fori_loop(0, NCHUNK // U1, p1, init)

    v1 = jnp.max(M1, axis=-1, keepdims=True)
    i1 = jnp.min(jnp.where(M1 == v1, I1, V), axis=-1, keepdims=True)
    cstar = I1 == i1
    candv = jnp.where(cstar, M2, M1)
    candi = jnp.where(cstar, I2, I1)
    v2 = jnp.max(candv, axis=-1, keepdims=True)
    i2 = jnp.min(jnp.where(candv == v2, candi, V), axis=-1, keepdims=True)

    ctail = x_ref[:, NCHUNK * CW:]
    tidx = lax.broadcasted_iota(jnp.int32, (R, TAILW), 1) + NCHUNK * CW
    m1, i1, m2, i2 = _merge_top2((v1, i1, v2, i2), ctail, tidx)

    def p2(j, S):
        cs = [x_ref[:, pl.ds(pl.multiple_of(j * (U2 * CWE) + u * CWE, CWE), CWE)]
              for u in range(U2)]
        for u in range(U2):
            S = S + jnp.exp(cs[u] - m1)
        return S

    S = lax.fori_loop(0, NCHUNKE // U2, p2, jnp.zeros((R, CWE), jnp.float32))
    s = jnp.sum(S, axis=-1, keepdims=True)
    s = s + jnp.sum(jnp.exp(x_ref[:, NCHUNKE * CWE:] - m1), axis=-1,
                    keepdims=True)
    lse = m1 + jnp.log(s)

    m1_ref[0, 0, :] = m1[:, 0]
    i1_ref[0, 0, :] = i1[:, 0]
    m2_ref[0, 0, :] = m2[:, 0]
    i2_ref[0, 0, :] = i2[:, 0]
    lse_ref[0, 0, :] = lse[:, 0]


def _row_stats(x):
    f32 = jax.ShapeDtypeStruct((NROWBLK, 1, R), jnp.float32)
    i32 = jax.ShapeDtypeStruct((NROWBLK, 1, R), jnp.int32)
    outs = pl.pallas_call(
        _stats_kernel,
        grid=(NROWBLK,),
        in_specs=[pl.BlockSpec((R, V), lambda i: (i, 0))],
        out_specs=[pl.BlockSpec((1, 1, R), lambda i: (i, 0, 0))] * 5,
        out_shape=[f32, i32, f32, i32, f32],
        compiler_params=pltpu.CompilerParams(
            dimension_semantics=("arbitrary",)),
    )(x)
    return [o.reshape(-1) for o in outs]


MASKNEG = jnp.float32(-1.0e38)


def _sc_tail_kernel(fpack_hbm, ipack_hbm,
                    oint_hbm, osc_hbm,
                    fv, iv, oint_v, osc_v, sem):
    b = lax.axis_index("s") * 2 + lax.axis_index("c")
    cf = pltpu.async_copy(fpack_hbm.at[b], fv, sem)
    ci = pltpu.async_copy(ipack_hbm.at[b], iv, sem)
    cf.wait()
    ci.wait()

    iota = lax.iota(jnp.int32, 16)

    def take(x, idx):
        return lax.gather(
            x, idx[:, None],
            lax.GatherDimensionNumbers(offset_dims=(),
                                       collapsed_slice_dims=(0,),
                                       start_index_map=(0,)),
            (1,), mode=lax.GatherScatterMode.PROMISE_IN_BOUNDS)

    def gsum(x):
        for sh in (4, 2, 1):
            x = x + take(x, iota ^ sh)
        return x

    def allmax(x):
        for sh in (8, 4, 2, 1):
            x = jnp.maximum(x, take(x, iota ^ sh))
        return x

    def allmin(x):
        for sh in (8, 4, 2, 1):
            x = jnp.minimum(x, take(x, iota ^ sh))
        return x

    seqA = iv[pl.ds(0, 16)]
    seqB = iv[pl.ds(16, 16)]
    scA = fv[pl.ds(0, 16)]
    scB = fv[pl.ds(16, 16)]
    stf = fv[pl.ds(32, 16)]
    sti = iv[pl.ds(32, 16)]

    lane03 = iota & 3
    half = (iota & 1) * 8

    psA = gsum(scA)
    psB = gsum(scB)
    pb = jnp.where(iota < 2, take(psA, half), take(psB, half))
    nzA = gsum(jnp.where(seqA != 0, 1, 0))
    nzB = gsum(jnp.where(seqB != 0, 1, 0))
    pnz = jnp.where(iota < 2, take(nzA, half), take(nzB, half))
    lastv = jnp.where(iota < 2, take(seqA, half + 7), take(seqB, half + 7))
    donei = jnp.where((lastv == 0) | (lastv == 2), 1, 0)

    done03 = donei == 1
    lseb = take(stf, lane03 + 8)
    logp1 = jnp.where(done03, 0.0, take(stf, lane03) - lseb)
    tok1 = jnp.where(done03, 0, take(sti, lane03))
    logp2 = jnp.where(done03, MASKNEG, take(stf, lane03 + 4) - lseb)
    tok2 = jnp.where(done03, 1, take(sti, lane03 + 4))

    wv = iota >> 1
    rv = iota & 1
    candlp = jnp.where(rv == 0, take(logp1, wv), take(logp2, wv))
    candtok = jnp.where(rv == 0, take(tok1, wv), take(tok2, wv))
    ps8 = take(pb, wv)
    pnz8 = take(pnz, wv)
    hyplen = pnz8 + jnp.where(candtok != 0, 1, 0)
    bscore = (ps8 + candlp) / ((5.0 + hyplen.astype(jnp.float32)) / 6.0)
    bscore = jnp.where(iota < 8, bscore, MASKNEG)

    sel = jnp.zeros((16,), jnp.int32)
    bs = bscore
    for k in range(4):
        mx = allmax(bs)
        selk = allmin(jnp.where(bs == mx, iota, 16))
        sel = jnp.where(iota == k, selk, sel)
        bs = jnp.where(iota == selk, MASKNEG, bs)

    srcw = sel >> 1
    selLogp = take(candlp, sel)
    selTok = take(candtok, sel)
    d2 = take(donei, srcw)
    ssum = take(pb, srcw)
    snz = take(pnz, srcw)
    lastok = jnp.where(d2 == 1, 0, selTok)
    outlen = snz + jnp.where(lastok != 0, 1, 0)
    fscore = (ssum + selLogp) / ((5.0 + outlen.astype(jnp.float32)) / 6.0)
    fscore = jnp.where(iota < 4, fscore, MASKNEG)

    sidx = jnp.zeros((16,), jnp.int32)
    fs = fscore
    for q in range(4):
        mx = allmax(fs)
        sq = allmin(jnp.where(fs == mx, iota, 16))
        sidx = jnp.where(iota == q, sq, sidx)
        fs = jnp.where(iota == sq, MASKNEG, fs)

    csrc = take(srcw, sidx)
    cscore = take(fscore, sidx)
    clen = take(outlen, sidx)
    clast = take(lastok, sidx)

    t8 = iota & 7
    qA = iota >> 3
    csA = take(csrc, qA)
    outA = jnp.where(csA < 2, take(seqA, (csA & 1) * 8 + t8),
                     take(seqB, (csA & 1) * 8 + t8))
    qB = qA + 2
    csB = take(csrc, qB)
    outB = jnp.where(csB < 2, take(seqA, (csB & 1) * 8 + t8),
                     take(seqB, (csB & 1) * 8 + t8))

    oint_v[pl.ds(0, 16)] = outA
    oint_v[pl.ds(16, 16)] = outB
    oint_v[pl.ds(32, 16)] = clast
    oint_v[pl.ds(48, 16)] = clen
    osc_v[...] = cscore

    co = pltpu.async_copy(oint_v, oint_hbm.at[b], sem)
    cs = pltpu.async_copy(osc_v, osc_hbm.at[b], sem)
    co.wait()
    cs.wait()


def _sc_tail(output_seq, scores, v1, i1, v2, i2, lse):
    B = output_seq.shape[0]
    seq2 = output_seq.reshape(B, W * L)
    sc2 = scores.reshape(B, W * L)
    fpack = jnp.concatenate([sc2, v1, v2, lse, jnp.zeros((B, 4), jnp.float32)],
                            axis=1)
    ipack = jnp.concatenate([seq2, i1, i2, jnp.zeros((B, 8), jnp.int32)],
                            axis=1)

    mesh = plsc.VectorSubcoreMesh(core_axis_name="c", subcore_axis_name="s")
    fn = functools.partial(
        pl.kernel,
        mesh=mesh,
        out_type=[
            jax.ShapeDtypeStruct((B, 64), jnp.int32),
            jax.ShapeDtypeStruct((B, 16), jnp.float32),
        ],
        scratch_types=[
            pltpu.VMEM((48,), jnp.float32),
            pltpu.VMEM((48,), jnp.int32),
            pltpu.VMEM((64,), jnp.int32),
            pltpu.VMEM((16,), jnp.float32),
            pltpu.SemaphoreType.DMA,
        ],
    )(_sc_tail_kernel)
    oint, osc = fn(fpack, ipack)
    out_seq = jnp.concatenate(
        [oint[:, :32].reshape(B, W, L), oint[:, 32:32 + W, None]], axis=2)
    return (out_seq, osc[:, :W], oint[:, 48:48 + W])


def kernel(new_logits, output_seq, scores):
    B = new_logits.shape[0]
    x = new_logits.reshape(B * W, V)
    m1, i1, m2, i2, lse = _row_stats(x)
    return _sc_tail(output_seq, scores, m1.reshape(B, W), i1.reshape(B, W),
                    m2.reshape(B, W), i2.reshape(B, W), lse.reshape(B, W))
